# Initial kernel scaffold; baseline (speedup 1.0000x reference)
import jax
import jax.numpy as jnp
from jax.experimental import pallas as pl


def kernel(x, edge_index, edge_attr, batch, W0, b0, We1, be1, We2, be2, root, cbias, gWih, gWhh, gbih, gbhh, lWih, lWhh, lbih, lbhh, W1, b1, W2, b2):
    raise NotImplementedError("write your pallas kernel here")


# speedup vs baseline: 1.5458x; 1.5458x over previous
import functools

import jax
import jax.numpy as jnp
from jax import lax
from jax.experimental import pallas as pl
from jax.experimental.pallas import tpu as pltpu
from jax.experimental.pallas import tpu_sc as plsc

_N = 4096
_E = 8192
_G = 256
_D = 64
_H = 128
_NUM_LAYER = 5
_STEPS = 3

_NW = 32
_EPW = _E // _NW
_CH = 128
_NCH = _EPW // _CH
_NPS = _N // 16

_f32 = jnp.float32


@functools.cache
def _build_sc_gather():
    mesh = plsc.VectorSubcoreMesh(core_axis_name="c", subcore_axis_name="s")

    @functools.partial(
        pl.kernel,
        out_type=jax.ShapeDtypeStruct((_E, _D), _f32),
        mesh=mesh,
        scratch_types=[
            pltpu.VMEM((_NCH, _CH), jnp.int32),
            pltpu.VMEM((_EPW, _D), _f32),
            pltpu.SemaphoreType.DMA,
        ],
        compiler_params=pltpu.CompilerParams(use_tc_tiling_on_sc=False),
    )
    def gather(h_hbm, src_hbm, out_hbm, idx_v, rows_v, sem):
        wid = lax.axis_index("s") * 2 + lax.axis_index("c")
        base = wid * _EPW
        for j in range(_NCH):
            pltpu.sync_copy(src_hbm.at[pl.ds(base + j * _CH, _CH)],
                            idx_v.at[j])
            pltpu.async_copy(h_hbm.at[idx_v.at[j]],
                             rows_v.at[pl.ds(j * _CH, _CH)], sem).wait()
        pltpu.sync_copy(rows_v, out_hbm.at[pl.ds(base, _EPW)])

    return gather


@functools.cache
def _build_sc_scatter_add():
    mesh = plsc.VectorSubcoreMesh(core_axis_name="c", subcore_axis_name="s")

    @functools.partial(
        pl.kernel,
        out_type=jax.ShapeDtypeStruct((2, _N, _D), _f32),
        mesh=mesh,
        scratch_types=[
            pltpu.VMEM((_NCH, _CH), jnp.int32),
            pltpu.VMEM((_EPW, _D), _f32),
            pltpu.VMEM_SHARED((_N, _D), _f32),
            pltpu.SemaphoreType.DMA,
        ],
        compiler_params=pltpu.CompilerParams(use_tc_tiling_on_sc=False),
    )
    def scatter_add(msg_hbm, dst_hbm, zero_hbm, out_hbm, idx_v, rows_v,
                    aggr_sh, sem):
        c = lax.axis_index("c")
        s = lax.axis_index("s")
        base = (s * 2 + c) * _EPW
        nb = s * _NPS
        pltpu.sync_copy(zero_hbm.at[pl.ds(nb, _NPS)],
                        aggr_sh.at[pl.ds(nb, _NPS)])
        pltpu.sync_copy(msg_hbm.at[pl.ds(base, _EPW)], rows_v)
        for j in range(_NCH):
            pltpu.sync_copy(dst_hbm.at[pl.ds(base + j * _CH, _CH)],
                            idx_v.at[j])
        plsc.subcore_barrier()
        for j in range(_NCH):
            pltpu.sync_copy(rows_v.at[pl.ds(j * _CH, _CH)],
                            aggr_sh.at[idx_v.at[j]], add=True)
        plsc.subcore_barrier()
        pltpu.sync_copy(aggr_sh.at[pl.ds(nb, _NPS)],
                        out_hbm.at[c, pl.ds(nb, _NPS)])

    return scatter_add


def _sc_gather(h, src):
    return _build_sc_gather()(h, src)


def _sc_scatter_add(msg, dst, zeros_n):
    return _build_sc_scatter_add()(msg, dst, zeros_n)


def _dotT(a, b):
    return lax.dot_general(a, b, (((1,), (1,)), ((), ())),
                           preferred_element_type=_f32)


def _pre_body(x_ref, w0_ref, b0_ref, ea_ref, we1_ref, be1_ref,
              h0_ref, hid_ref):
    h0 = _dotT(x_ref[...], w0_ref[...]) + b0_ref[...]
    h0_ref[...] = jnp.maximum(h0, 0.0)
    hid = _dotT(ea_ref[...], we1_ref[...]) + be1_ref[...]
    hid_ref[...] = jnp.maximum(hid, 0.0)


_EB = 512


def _msg_body(hid_ref, xs_ref, we2_ref, be2_ref, msg_ref):
    w = _dotT(hid_ref[...], we2_ref[...]) + be2_ref[...]
    xs = xs_ref[...]
    w3 = w.reshape(_EB, _D, _D)
    msg_ref[...] = jnp.sum(w3 * xs[:, :, None], axis=1)


def _update_body(ap_ref, dp_ref, h_ref, root_ref, cb_ref, gwih_ref, gwhh_ref,
                 gbih_ref, gbhh_ref, out_ref):
    aggr = ap_ref[0] + ap_ref[1]
    deg = jnp.maximum(dp_ref[0] + dp_ref[1], 1.0)
    h = h_ref[...]
    m = jnp.maximum(aggr / deg
                    + jnp.dot(h, root_ref[...], preferred_element_type=_f32)
                    + cb_ref[...], 0.0)
    gi = _dotT(m, gwih_ref[...]) + gbih_ref[...]
    gh = _dotT(h, gwhh_ref[...]) + gbhh_ref[...]
    r = jax.nn.sigmoid(gi[:, :_D] + gh[:, :_D])
    z = jax.nn.sigmoid(gi[:, _D:2 * _D] + gh[:, _D:2 * _D])
    n = jnp.tanh(gi[:, 2 * _D:] + r * gh[:, 2 * _D:])
    out_ref[...] = (1.0 - z) * n + z * h


def _s2s_body(h_ref, batch_ref, lwih_ref, lwhh_ref, lbih_ref, lbhh_ref,
              w1_ref, b1_ref, w2_ref, b2_ref, out_ref):
    out = h_ref[...]
    seg = lax.broadcasted_iota(jnp.int32, (_N, _G), 1)
    mb = batch_ref[...] == seg
    mf = jnp.where(mb, 1.0, 0.0)
    q_star = jnp.zeros((_G, 2 * _D), _f32)
    hh = jnp.zeros((_G, _D), _f32)
    cc = jnp.zeros((_G, _D), _f32)
    for _ in range(_STEPS):
        g = (_dotT(q_star, lwih_ref[...]) + lbih_ref[...]
             + _dotT(hh, lwhh_ref[...]) + lbhh_ref[...])
        ig = jax.nn.sigmoid(g[:, :_D])
        fg = jax.nn.sigmoid(g[:, _D:2 * _D])
        gg = jnp.tanh(g[:, 2 * _D:3 * _D])
        og = jax.nn.sigmoid(g[:, 3 * _D:])
        cc = fg * cc + ig * gg
        hh = og * jnp.tanh(cc)
        qb = jnp.dot(mf, hh, preferred_element_type=_f32)
        e = jnp.sum(out * qb, axis=1, keepdims=True)
        em = jnp.where(mb, e, -jnp.inf)
        emax = jnp.max(em, axis=0, keepdims=True)
        emax = jnp.where(jnp.isfinite(emax), emax, 0.0)
        eshift = e - jnp.sum(mf * emax, axis=1, keepdims=True)
        ew = jnp.exp(eshift)
        denom = lax.dot_general(mf, ew, (((0,), (0,)), ((), ())),
                                preferred_element_type=_f32)
        rden = 1.0 / jnp.maximum(denom, 1e-16)
        a = ew * jnp.dot(mf, rden, preferred_element_type=_f32)
        rvec = lax.dot_general(mf, a * out, (((0,), (0,)), ((), ())),
                               preferred_element_type=_f32)
        q_star = jnp.concatenate([hh, rvec], axis=1)
    o1 = _dotT(q_star, w1_ref[...]) + b1_ref[...]
    o2 = _dotT(o1, w2_ref[...]) + b2_ref[...]
    out_ref[...] = jnp.where(o2 > 0.0, o2, 0.01 * o2)


def kernel(x, edge_index, edge_attr, batch, W0, b0, We1, be1, We2, be2, root,
           cbias, gWih, gWhh, gbih, gbhh, lWih, lWhh, lbih, lbhh, W1, b1,
           W2, b2):
    src = edge_index[0]
    dst = edge_index[1]

    h0, hidden = pl.pallas_call(
        _pre_body,
        out_shape=(jax.ShapeDtypeStruct((_N, _D), _f32),
                   jax.ShapeDtypeStruct((_E, _H), _f32)),
    )(x, W0, b0.reshape(1, -1), edge_attr, We1, be1.reshape(1, -1))

    zeros_n = jnp.zeros((_N, _D), _f32)
    ones_e = jnp.ones((_E, _D), _f32)
    deg_parts = _sc_scatter_add(ones_e, dst, zeros_n)

    msg_call = pl.pallas_call(
        _msg_body,
        grid=(_E // _EB,),
        in_specs=[
            pl.BlockSpec((_EB, _H), lambda i: (i, 0)),
            pl.BlockSpec((_EB, _D), lambda i: (i, 0)),
            pl.BlockSpec((_D * _D, _H), lambda i: (0, 0)),
            pl.BlockSpec((1, _D * _D), lambda i: (0, 0)),
        ],
        out_specs=pl.BlockSpec((_EB, _D), lambda i: (i, 0)),
        out_shape=jax.ShapeDtypeStruct((_E, _D), _f32),
    )
    update_call = pl.pallas_call(
        _update_body,
        out_shape=jax.ShapeDtypeStruct((_N, _D), _f32),
    )

    be2r = be2.reshape(1, -1)
    h = h0
    for _ in range(_NUM_LAYER):
        xs = _sc_gather(h, src)
        msg = msg_call(hidden, xs, We2, be2r)
        aggr_parts = _sc_scatter_add(msg, dst, zeros_n)
        h = update_call(aggr_parts, deg_parts, h, root,
                        cbias.reshape(1, -1), gWih, gWhh,
                        gbih.reshape(1, -1), gbhh.reshape(1, -1))

    return pl.pallas_call(
        _s2s_body,
        out_shape=jax.ShapeDtypeStruct((_G, 300), _f32),
    )(h, batch.reshape(-1, 1), lWih, lWhh, lbih.reshape(1, -1),
      lbhh.reshape(1, -1), W1, b1.reshape(1, -1), W2, b2.reshape(1, -1))


# speedup vs baseline: 1.5753x; 1.0191x over previous
import functools

import jax
import jax.numpy as jnp
from jax import lax
from jax.experimental import pallas as pl
from jax.experimental.pallas import tpu as pltpu
from jax.experimental.pallas import tpu_sc as plsc

_N = 4096
_E = 8192
_G = 256
_D = 64
_H = 128
_NUM_LAYER = 5
_STEPS = 3

_NW = 32
_EPW = _E // _NW
_CH = 128
_NCH = _EPW // _CH
_NPS = _N // 16

_f32 = jnp.float32


@functools.cache
def _build_sc_gather():
    mesh = plsc.VectorSubcoreMesh(core_axis_name="c", subcore_axis_name="s")

    @functools.partial(
        pl.kernel,
        out_type=jax.ShapeDtypeStruct((_E, _D), _f32),
        mesh=mesh,
        scratch_types=[
            pltpu.VMEM((_NCH, _CH), jnp.int32),
            pltpu.VMEM((_EPW, _D), _f32),
            pltpu.SemaphoreType.DMA,
        ],
        compiler_params=pltpu.CompilerParams(use_tc_tiling_on_sc=False),
    )
    def gather(h_hbm, ei_hbm, out_hbm, idx_v, rows_v, sem):
        wid = lax.axis_index("s") * 2 + lax.axis_index("c")
        base = wid * _EPW
        for j in range(_NCH):
            pltpu.sync_copy(ei_hbm.at[0, pl.ds(base + j * _CH, _CH)],
                            idx_v.at[j])
            pltpu.async_copy(h_hbm.at[idx_v.at[j]],
                             rows_v.at[pl.ds(j * _CH, _CH)], sem).wait()
        pltpu.sync_copy(rows_v, out_hbm.at[pl.ds(base, _EPW)])

    return gather


@functools.cache
def _build_sc_scatter_add():
    mesh = plsc.VectorSubcoreMesh(core_axis_name="c", subcore_axis_name="s")

    @functools.partial(
        pl.kernel,
        out_type=jax.ShapeDtypeStruct((2, _N, _D), _f32),
        mesh=mesh,
        scratch_types=[
            pltpu.VMEM((_NCH, _CH), jnp.int32),
            pltpu.VMEM((_EPW, _D), _f32),
            pltpu.VMEM_SHARED((_N, _D), _f32),
            pltpu.SemaphoreType.DMA,
        ],
        compiler_params=pltpu.CompilerParams(use_tc_tiling_on_sc=False),
    )
    def scatter_add(msg_hbm, ei_hbm, zero_hbm, out_hbm, idx_v, rows_v,
                    aggr_sh, sem):
        c = lax.axis_index("c")
        s = lax.axis_index("s")
        base = (s * 2 + c) * _EPW
        nb = s * _NPS
        pltpu.sync_copy(zero_hbm.at[pl.ds(nb, _NPS)],
                        aggr_sh.at[pl.ds(nb, _NPS)])
        pltpu.sync_copy(msg_hbm.at[pl.ds(base, _EPW)], rows_v)
        for j in range(_NCH):
            pltpu.sync_copy(ei_hbm.at[1, pl.ds(base + j * _CH, _CH)],
                            idx_v.at[j])
        plsc.subcore_barrier()
        for j in range(_NCH):
            pltpu.sync_copy(rows_v.at[pl.ds(j * _CH, _CH)],
                            aggr_sh.at[idx_v.at[j]], add=True)
        plsc.subcore_barrier()
        pltpu.sync_copy(aggr_sh.at[pl.ds(nb, _NPS)],
                        out_hbm.at[c, pl.ds(nb, _NPS)])

    return scatter_add


def _sc_gather(h, edge_index):
    return _build_sc_gather()(h, edge_index)


def _sc_scatter_add(msg, edge_index, zeros_n):
    return _build_sc_scatter_add()(msg, edge_index, zeros_n)


def _dotT(a, b):
    return lax.dot_general(a, b, (((1,), (1,)), ((), ())),
                           preferred_element_type=_f32)


def _pre_body(x_ref, w0_ref, b0_ref, ea_ref, we1_ref, be1_ref,
              h0_ref, hid_ref):
    h0 = _dotT(x_ref[...], w0_ref[...]) + b0_ref[...][None, :]
    h0_ref[...] = jnp.maximum(h0, 0.0)
    hid = _dotT(ea_ref[...], we1_ref[...]) + be1_ref[...][None, :]
    hid_ref[...] = jnp.maximum(hid, 0.0).astype(jnp.bfloat16)


_EB = 512


def _msg_body(hid_ref, xs_ref, we2_ref, be2_ref, msg_ref):
    w = _dotT(hid_ref[...], we2_ref[...]) + be2_ref[...][None, :]
    xs = xs_ref[...]
    w3 = w.reshape(_EB, _D, _D)
    msg_ref[...] = jnp.sum(w3 * xs[:, :, None], axis=1)


def _update_body(ap_ref, dp_ref, h_ref, root_ref, cb_ref, gwih_ref, gwhh_ref,
                 gbih_ref, gbhh_ref, out_ref):
    aggr = ap_ref[0] + ap_ref[1]
    deg = jnp.maximum(dp_ref[0] + dp_ref[1], 1.0)
    h = h_ref[...]
    m = jnp.maximum(aggr / deg
                    + jnp.dot(h, root_ref[...], preferred_element_type=_f32)
                    + cb_ref[...][None, :], 0.0)
    gi = _dotT(m, gwih_ref[...]) + gbih_ref[...][None, :]
    gh = _dotT(h, gwhh_ref[...]) + gbhh_ref[...][None, :]
    r = jax.nn.sigmoid(gi[:, :_D] + gh[:, :_D])
    z = jax.nn.sigmoid(gi[:, _D:2 * _D] + gh[:, _D:2 * _D])
    n = jnp.tanh(gi[:, 2 * _D:] + r * gh[:, 2 * _D:])
    out_ref[...] = (1.0 - z) * n + z * h


def _s2s_body(h_ref, batch_ref, lwih_ref, lwhh_ref, lbih_ref, lbhh_ref,
              w1_ref, b1_ref, w2_ref, b2_ref, out_ref):
    out = h_ref[...]
    seg = lax.broadcasted_iota(jnp.int32, (_N, _G), 1)
    mb = batch_ref[...][:, None] == seg
    mf = jnp.where(mb, 1.0, 0.0)
    q_star = jnp.zeros((_G, 2 * _D), _f32)
    hh = jnp.zeros((_G, _D), _f32)
    cc = jnp.zeros((_G, _D), _f32)
    for _ in range(_STEPS):
        g = (_dotT(q_star, lwih_ref[...]) + lbih_ref[...][None, :]
             + _dotT(hh, lwhh_ref[...]) + lbhh_ref[...][None, :])
        ig = jax.nn.sigmoid(g[:, :_D])
        fg = jax.nn.sigmoid(g[:, _D:2 * _D])
        gg = jnp.tanh(g[:, 2 * _D:3 * _D])
        og = jax.nn.sigmoid(g[:, 3 * _D:])
        cc = fg * cc + ig * gg
        hh = og * jnp.tanh(cc)
        qb = jnp.dot(mf, hh, preferred_element_type=_f32)
        e = jnp.sum(out * qb, axis=1, keepdims=True)
        em = jnp.where(mb, e, -jnp.inf)
        emax = jnp.max(em, axis=0, keepdims=True)
        emax = jnp.where(jnp.isfinite(emax), emax, 0.0)
        eshift = e - jnp.sum(mf * emax, axis=1, keepdims=True)
        ew = jnp.exp(eshift)
        denom = lax.dot_general(mf, ew, (((0,), (0,)), ((), ())),
                                preferred_element_type=_f32)
        rden = 1.0 / jnp.maximum(denom, 1e-16)
        a = ew * jnp.dot(mf, rden, preferred_element_type=_f32)
        rvec = lax.dot_general(mf, a * out, (((0,), (0,)), ((), ())),
                               preferred_element_type=_f32)
        q_star = jnp.concatenate([hh, rvec], axis=1)
    o1 = _dotT(q_star, w1_ref[...]) + b1_ref[...][None, :]
    o2 = _dotT(o1, w2_ref[...]) + b2_ref[...][None, :]
    out_ref[...] = jnp.where(o2 > 0.0, o2, 0.01 * o2)


def kernel(x, edge_index, edge_attr, batch, W0, b0, We1, be1, We2, be2, root,
           cbias, gWih, gWhh, gbih, gbhh, lWih, lWhh, lbih, lbhh, W1, b1,
           W2, b2):
    h0, hidden = pl.pallas_call(
        _pre_body,
        out_shape=(jax.ShapeDtypeStruct((_N, _D), _f32),
                   jax.ShapeDtypeStruct((_E, _H), jnp.bfloat16)),
    )(x, W0, b0, edge_attr, We1, be1)

    zeros_n = jnp.zeros((_N, _D), _f32)
    ones_e = jnp.ones((_E, _D), _f32)
    deg_parts = _sc_scatter_add(ones_e, edge_index, zeros_n)

    msg_call = pl.pallas_call(
        _msg_body,
        grid=(_E // _EB,),
        in_specs=[
            pl.BlockSpec((_EB, _H), lambda i: (i, 0)),
            pl.BlockSpec((_EB, _D), lambda i: (i, 0)),
            pl.BlockSpec((_D * _D, _H), lambda i: (0, 0)),
            pl.BlockSpec((_D * _D,), lambda i: (0,)),
        ],
        out_specs=pl.BlockSpec((_EB, _D), lambda i: (i, 0)),
        out_shape=jax.ShapeDtypeStruct((_E, _D), _f32),
    )
    update_call = pl.pallas_call(
        _update_body,
        out_shape=jax.ShapeDtypeStruct((_N, _D), _f32),
    )

    we2b = We2.astype(jnp.bfloat16)
    h = h0
    for _ in range(_NUM_LAYER):
        xs = _sc_gather(h, edge_index)
        msg = msg_call(hidden, xs, we2b, be2)
        aggr_parts = _sc_scatter_add(msg, edge_index, zeros_n)
        h = update_call(aggr_parts, deg_parts, h, root, cbias,
                        gWih, gWhh, gbih, gbhh)

    return pl.pallas_call(
        _s2s_body,
        out_shape=jax.ShapeDtypeStruct((_G, 300), _f32),
    )(h, batch, lWih, lWhh, lbih, lbhh, W1, b1, W2, b2)


# speedup vs baseline: 3.5516x; 2.2546x over previous
import functools

import jax
import jax.numpy as jnp
from jax import lax
from jax.experimental import pallas as pl
from jax.experimental.pallas import tpu as pltpu
from jax.experimental.pallas import tpu_sc as plsc

_N = 4096
_E = 8192
_G = 256
_D = 64
_H = 128
_NUM_LAYER = 5
_STEPS = 3

_NW = 32
_EPW = _E // _NW
_CH = 128
_NCH = _EPW // _CH
_NPS = _N // 16

_f32 = jnp.float32


@functools.cache
def _build_sc_gather():
    mesh = plsc.VectorSubcoreMesh(core_axis_name="c", subcore_axis_name="s")

    @functools.partial(
        pl.kernel,
        out_type=jax.ShapeDtypeStruct((_E, _D), _f32),
        mesh=mesh,
        scratch_types=[
            pltpu.VMEM((_NCH, _CH), jnp.int32),
            pltpu.VMEM((_EPW, _D), _f32),
            pltpu.SemaphoreType.DMA,
        ],
        compiler_params=pltpu.CompilerParams(use_tc_tiling_on_sc=False),
    )
    def gather(h_hbm, ei_hbm, out_hbm, idx_v, rows_v, sem):
        wid = lax.axis_index("s") * 2 + lax.axis_index("c")
        base = wid * _EPW
        for j in range(_NCH):
            pltpu.sync_copy(ei_hbm.at[0, pl.ds(base + j * _CH, _CH)],
                            idx_v.at[j])
            pltpu.async_copy(h_hbm.at[idx_v.at[j]],
                             rows_v.at[pl.ds(j * _CH, _CH)], sem).wait()
        pltpu.sync_copy(rows_v, out_hbm.at[pl.ds(base, _EPW)])

    return gather


@functools.cache
def _build_sc_scatter_add():
    mesh = plsc.VectorSubcoreMesh(core_axis_name="c", subcore_axis_name="s")

    @functools.partial(
        pl.kernel,
        out_type=jax.ShapeDtypeStruct((2, _N, _D), _f32),
        mesh=mesh,
        scratch_types=[
            pltpu.VMEM((_NCH, _CH), jnp.int32),
            pltpu.VMEM((_EPW, _D), _f32),
            pltpu.VMEM_SHARED((_N, _D), _f32),
            pltpu.SemaphoreType.DMA,
        ],
        compiler_params=pltpu.CompilerParams(use_tc_tiling_on_sc=False),
    )
    def scatter_add(msg_hbm, ei_hbm, zero_hbm, out_hbm, idx_v, rows_v,
                    aggr_sh, sem):
        c = lax.axis_index("c")
        s = lax.axis_index("s")
        base = (s * 2 + c) * _EPW
        nb = s * _NPS
        pltpu.sync_copy(zero_hbm.at[pl.ds(nb, _NPS)],
                        aggr_sh.at[pl.ds(nb, _NPS)])
        pltpu.sync_copy(msg_hbm.at[pl.ds(base, _EPW)], rows_v)
        for j in range(_NCH):
            pltpu.sync_copy(ei_hbm.at[1, pl.ds(base + j * _CH, _CH)],
                            idx_v.at[j])
        plsc.subcore_barrier()
        for j in range(_NCH):
            pltpu.sync_copy(rows_v.at[pl.ds(j * _CH, _CH)],
                            aggr_sh.at[idx_v.at[j]], add=True)
        plsc.subcore_barrier()
        pltpu.sync_copy(aggr_sh.at[pl.ds(nb, _NPS)],
                        out_hbm.at[c, pl.ds(nb, _NPS)])

    return scatter_add


def _sc_gather(h, edge_index):
    return _build_sc_gather()(h, edge_index)


def _sc_scatter_add(msg, edge_index, zeros_n):
    return _build_sc_scatter_add()(msg, edge_index, zeros_n)


def _dotT(a, b):
    return lax.dot_general(a, b, (((1,), (1,)), ((), ())),
                           preferred_element_type=_f32)


def _pre_body(x_ref, w0_ref, b0_ref, ea_ref, we1_ref, be1_ref,
              h0_ref, hid_ref):
    h0 = _dotT(x_ref[...], w0_ref[...]) + b0_ref[...][None, :]
    h0_ref[...] = jnp.maximum(h0, 0.0)
    hid = _dotT(ea_ref[...], we1_ref[...]) + be1_ref[...][None, :]
    hid_ref[...] = jnp.maximum(hid, 0.0).astype(jnp.bfloat16)


_EB = 512


def _msg_body(hid_ref, xs_ref, we2_ref, be2m_ref, msg_ref):
    wt = lax.dot_general(we2_ref[...], hid_ref[...], (((1,), (1,)), ((), ())),
                         preferred_element_type=_f32)
    xs = xs_ref[...]
    xst = xs.T
    acc = wt[0:_D, :] * xst[0:1, :]
    for i in range(1, _D):
        acc = acc + wt[i * _D:(i + 1) * _D, :] * xst[i:i + 1, :]
    msg_ref[...] = acc.T + jnp.dot(xs, be2m_ref[...],
                                   preferred_element_type=_f32)


def _update_body(ap_ref, dp_ref, h_ref, root_ref, cb_ref, gwih_ref, gwhh_ref,
                 gbih_ref, gbhh_ref, out_ref):
    aggr = ap_ref[0] + ap_ref[1]
    deg = jnp.maximum(dp_ref[0] + dp_ref[1], 1.0)
    h = h_ref[...]
    m = jnp.maximum(aggr / deg
                    + jnp.dot(h, root_ref[...], preferred_element_type=_f32)
                    + cb_ref[...][None, :], 0.0)
    gi = _dotT(m, gwih_ref[...]) + gbih_ref[...][None, :]
    gh = _dotT(h, gwhh_ref[...]) + gbhh_ref[...][None, :]
    r = jax.nn.sigmoid(gi[:, :_D] + gh[:, :_D])
    z = jax.nn.sigmoid(gi[:, _D:2 * _D] + gh[:, _D:2 * _D])
    n = jnp.tanh(gi[:, 2 * _D:] + r * gh[:, 2 * _D:])
    out_ref[...] = (1.0 - z) * n + z * h


def _s2s_body(h_ref, batch_ref, lwih_ref, lwhh_ref, lbih_ref, lbhh_ref,
              w1_ref, b1_ref, w2_ref, b2_ref, out_ref):
    out = h_ref[...]
    seg = lax.broadcasted_iota(jnp.int32, (_N, _G), 1)
    mb = batch_ref[...][:, None] == seg
    mf = jnp.where(mb, 1.0, 0.0)
    q_star = jnp.zeros((_G, 2 * _D), _f32)
    hh = jnp.zeros((_G, _D), _f32)
    cc = jnp.zeros((_G, _D), _f32)
    for _ in range(_STEPS):
        g = (_dotT(q_star, lwih_ref[...]) + lbih_ref[...][None, :]
             + _dotT(hh, lwhh_ref[...]) + lbhh_ref[...][None, :])
        ig = jax.nn.sigmoid(g[:, :_D])
        fg = jax.nn.sigmoid(g[:, _D:2 * _D])
        gg = jnp.tanh(g[:, 2 * _D:3 * _D])
        og = jax.nn.sigmoid(g[:, 3 * _D:])
        cc = fg * cc + ig * gg
        hh = og * jnp.tanh(cc)
        qb = jnp.dot(mf, hh, preferred_element_type=_f32)
        e = jnp.sum(out * qb, axis=1, keepdims=True)
        em = jnp.where(mb, e, -jnp.inf)
        emax = jnp.max(em, axis=0, keepdims=True)
        emax = jnp.where(jnp.isfinite(emax), emax, 0.0)
        eshift = e - jnp.sum(mf * emax, axis=1, keepdims=True)
        ew = jnp.exp(eshift)
        denom = lax.dot_general(mf, ew, (((0,), (0,)), ((), ())),
                                preferred_element_type=_f32)
        rden = 1.0 / jnp.maximum(denom, 1e-16)
        a = ew * jnp.dot(mf, rden, preferred_element_type=_f32)
        rvec = lax.dot_general(mf, a * out, (((0,), (0,)), ((), ())),
                               preferred_element_type=_f32)
        q_star = jnp.concatenate([hh, rvec], axis=1)
    o1 = _dotT(q_star, w1_ref[...]) + b1_ref[...][None, :]
    o2 = _dotT(o1, w2_ref[...]) + b2_ref[...][None, :]
    out_ref[...] = jnp.where(o2 > 0.0, o2, 0.01 * o2)


def kernel(x, edge_index, edge_attr, batch, W0, b0, We1, be1, We2, be2, root,
           cbias, gWih, gWhh, gbih, gbhh, lWih, lWhh, lbih, lbhh, W1, b1,
           W2, b2):
    h0, hidden = pl.pallas_call(
        _pre_body,
        out_shape=(jax.ShapeDtypeStruct((_N, _D), _f32),
                   jax.ShapeDtypeStruct((_E, _H), jnp.bfloat16)),
    )(x, W0, b0, edge_attr, We1, be1)

    zeros_n = jnp.zeros((_N, _D), _f32)
    ones_e = jnp.ones((_E, _D), _f32)
    deg_parts = _sc_scatter_add(ones_e, edge_index, zeros_n)

    msg_call = pl.pallas_call(
        _msg_body,
        grid=(_E // _EB,),
        in_specs=[
            pl.BlockSpec((_EB, _H), lambda i: (i, 0)),
            pl.BlockSpec((_EB, _D), lambda i: (i, 0)),
            pl.BlockSpec((_D * _D, _H), lambda i: (0, 0)),
            pl.BlockSpec((_D, _D), lambda i: (0, 0)),
        ],
        out_specs=pl.BlockSpec((_EB, _D), lambda i: (i, 0)),
        out_shape=jax.ShapeDtypeStruct((_E, _D), _f32),
    )
    update_call = pl.pallas_call(
        _update_body,
        out_shape=jax.ShapeDtypeStruct((_N, _D), _f32),
    )

    we2b = We2.astype(jnp.bfloat16)
    be2m = be2.reshape(_D, _D)
    h = h0
    for _ in range(_NUM_LAYER):
        xs = _sc_gather(h, edge_index)
        msg = msg_call(hidden, xs, we2b, be2m)
        aggr_parts = _sc_scatter_add(msg, edge_index, zeros_n)
        h = update_call(aggr_parts, deg_parts, h, root, cbias,
                        gWih, gWhh, gbih, gbhh)

    return pl.pallas_call(
        _s2s_body,
        out_shape=jax.ShapeDtypeStruct((_G, 300), _f32),
    )(h, batch, lWih, lWhh, lbih, lbhh, W1, b1, W2, b2)


# speedup vs baseline: 4.3061x; 1.2124x over previous
import functools

import jax
import jax.numpy as jnp
from jax import lax
from jax.experimental import pallas as pl
from jax.experimental.pallas import tpu as pltpu
from jax.experimental.pallas import tpu_sc as plsc

_N = 4096
_E = 8192
_G = 256
_D = 64
_H = 128
_NUM_LAYER = 5
_STEPS = 3

_NW = 32
_EPW = _E // _NW
_CH = 128
_NCH = _EPW // _CH
_NPS = _N // 16
_DP = 128

_f32 = jnp.float32


@functools.cache
def _build_sc_gather():
    mesh = plsc.VectorSubcoreMesh(core_axis_name="c", subcore_axis_name="s")

    @functools.partial(
        pl.kernel,
        out_type=jax.ShapeDtypeStruct((_E, _DP), _f32),
        mesh=mesh,
        scratch_types=[
            pltpu.VMEM((_NCH, _CH), jnp.int32),
            pltpu.VMEM((_EPW, _DP), _f32),
            pltpu.SemaphoreType.DMA,
        ],
    )
    def gather(h_hbm, src_hbm, out_hbm, idx_v, rows_v, sem):
        wid = lax.axis_index("s") * 2 + lax.axis_index("c")
        base = wid * _EPW
        for j in range(_NCH):
            pltpu.sync_copy(src_hbm.at[pl.ds(base + j * _CH, _CH)],
                            idx_v.at[j])
            pltpu.async_copy(h_hbm.at[idx_v.at[j]],
                             rows_v.at[pl.ds(j * _CH, _CH)], sem).wait()
        pltpu.sync_copy(rows_v, out_hbm.at[pl.ds(base, _EPW)])

    return gather


@functools.cache
def _build_sc_scatter_add():
    mesh = plsc.VectorSubcoreMesh(core_axis_name="c", subcore_axis_name="s")

    @functools.partial(
        pl.kernel,
        out_type=jax.ShapeDtypeStruct((2, _N, _DP), _f32),
        mesh=mesh,
        scratch_types=[
            pltpu.VMEM((_NCH, _CH), jnp.int32),
            pltpu.VMEM((_EPW, _DP), _f32),
            pltpu.VMEM_SHARED((_N, _DP), _f32),
            pltpu.SemaphoreType.DMA,
        ],
    )
    def scatter_add(msg_hbm, dst_hbm, zero_hbm, out_hbm, idx_v, rows_v,
                    aggr_sh, sem):
        c = lax.axis_index("c")
        s = lax.axis_index("s")
        base = (s * 2 + c) * _EPW
        nb = s * _NPS
        pltpu.sync_copy(zero_hbm.at[pl.ds(nb, _NPS)],
                        aggr_sh.at[pl.ds(nb, _NPS)])
        pltpu.sync_copy(msg_hbm.at[pl.ds(base, _EPW)], rows_v)
        for j in range(_NCH):
            pltpu.sync_copy(dst_hbm.at[pl.ds(base + j * _CH, _CH)],
                            idx_v.at[j])
        plsc.subcore_barrier()
        for j in range(_NCH):
            pltpu.sync_copy(rows_v.at[pl.ds(j * _CH, _CH)],
                            aggr_sh.at[idx_v.at[j]], add=True)
        plsc.subcore_barrier()
        pltpu.sync_copy(aggr_sh.at[pl.ds(nb, _NPS)],
                        out_hbm.at[c, pl.ds(nb, _NPS)])

    return scatter_add


def _sc_gather(h, src):
    return _build_sc_gather()(h, src)


def _sc_scatter_add(msg, dst, zeros_n):
    return _build_sc_scatter_add()(msg, dst, zeros_n)


def _dotT(a, b):
    return lax.dot_general(a, b, (((1,), (1,)), ((), ())),
                           preferred_element_type=_f32)


def _pre_body(x_ref, w0_ref, b0_ref, ea_ref, we1_ref, be1_ref, we2_ref,
              h0_ref, hid_ref, we2b_ref):
    h0 = _dotT(x_ref[...], w0_ref[...]) + b0_ref[...][None, :]
    h0_ref[...] = jnp.concatenate(
        [jnp.maximum(h0, 0.0), jnp.zeros((_N, _DP - _D), _f32)], axis=1)
    hid = _dotT(ea_ref[...], we1_ref[...]) + be1_ref[...][None, :]
    hid_ref[...] = jnp.maximum(hid, 0.0).astype(jnp.bfloat16)
    we2b_ref[...] = we2_ref[...].astype(jnp.bfloat16)


_EB = 512


def _msg_body(hid_ref, xs_ref, we2_ref, be2m_ref, msg_ref):
    wt = lax.dot_general(we2_ref[...], hid_ref[...], (((1,), (1,)), ((), ())),
                         preferred_element_type=_f32)
    xs = xs_ref[...][:, 0:_D]
    xst = xs.T
    acc = wt[0:_D, :] * xst[0:1, :]
    for i in range(1, _D):
        acc = acc + wt[i * _D:(i + 1) * _D, :] * xst[i:i + 1, :]
    msg = acc.T + jnp.dot(xs, be2m_ref[...], preferred_element_type=_f32)
    msg_ref[...] = jnp.concatenate(
        [msg, jnp.zeros((_EB, _DP - _D), _f32)], axis=1)


def _update_body(ap_ref, dp_ref, h_ref, root_ref, cb_ref, gwih_ref, gwhh_ref,
                 gbih_ref, gbhh_ref, out_ref):
    aggr = ap_ref[0][:, 0:_D] + ap_ref[1][:, 0:_D]
    deg = jnp.maximum(dp_ref[0][:, 0:_D] + dp_ref[1][:, 0:_D], 1.0)
    h = h_ref[...][:, 0:_D]
    m = jnp.maximum(aggr / deg
                    + jnp.dot(h, root_ref[...], preferred_element_type=_f32)
                    + cb_ref[...][None, :], 0.0)
    gi = _dotT(m, gwih_ref[...]) + gbih_ref[...][None, :]
    gh = _dotT(h, gwhh_ref[...]) + gbhh_ref[...][None, :]
    r = jax.nn.sigmoid(gi[:, :_D] + gh[:, :_D])
    z = jax.nn.sigmoid(gi[:, _D:2 * _D] + gh[:, _D:2 * _D])
    n = jnp.tanh(gi[:, 2 * _D:] + r * gh[:, 2 * _D:])
    out_ref[...] = jnp.concatenate(
        [(1.0 - z) * n + z * h, jnp.zeros((_N, _DP - _D), _f32)], axis=1)


def _s2s_body(h_ref, batch_ref, lwih_ref, lwhh_ref, lbih_ref, lbhh_ref,
              w1_ref, b1_ref, w2_ref, b2_ref, out_ref):
    out = h_ref[...][:, 0:_D]
    seg = lax.broadcasted_iota(jnp.int32, (_N, _G), 1)
    mb = batch_ref[...][:, None] == seg
    mf = jnp.where(mb, 1.0, 0.0)
    q_star = jnp.zeros((_G, 2 * _D), _f32)
    hh = jnp.zeros((_G, _D), _f32)
    cc = jnp.zeros((_G, _D), _f32)
    for _ in range(_STEPS):
        g = (_dotT(q_star, lwih_ref[...]) + lbih_ref[...][None, :]
             + _dotT(hh, lwhh_ref[...]) + lbhh_ref[...][None, :])
        ig = jax.nn.sigmoid(g[:, :_D])
        fg = jax.nn.sigmoid(g[:, _D:2 * _D])
        gg = jnp.tanh(g[:, 2 * _D:3 * _D])
        og = jax.nn.sigmoid(g[:, 3 * _D:])
        cc = fg * cc + ig * gg
        hh = og * jnp.tanh(cc)
        qb = jnp.dot(mf, hh, preferred_element_type=_f32)
        e = jnp.sum(out * qb, axis=1, keepdims=True)
        em = jnp.where(mb, e, -jnp.inf)
        emax = jnp.max(em, axis=0, keepdims=True)
        emax = jnp.where(jnp.isfinite(emax), emax, 0.0)
        eshift = e - jnp.sum(mf * emax, axis=1, keepdims=True)
        ew = jnp.exp(eshift)
        denom = lax.dot_general(mf, ew, (((0,), (0,)), ((), ())),
                                preferred_element_type=_f32)
        rden = 1.0 / jnp.maximum(denom, 1e-16)
        a = ew * jnp.dot(mf, rden, preferred_element_type=_f32)
        rvec = lax.dot_general(mf, a * out, (((0,), (0,)), ((), ())),
                               preferred_element_type=_f32)
        q_star = jnp.concatenate([hh, rvec], axis=1)
    o1 = _dotT(q_star, w1_ref[...]) + b1_ref[...][None, :]
    o2 = _dotT(o1, w2_ref[...]) + b2_ref[...][None, :]
    out_ref[...] = jnp.where(o2 > 0.0, o2, 0.01 * o2)


def kernel(x, edge_index, edge_attr, batch, W0, b0, We1, be1, We2, be2, root,
           cbias, gWih, gWhh, gbih, gbhh, lWih, lWhh, lbih, lbhh, W1, b1,
           W2, b2):
    src_idx = edge_index[0]
    dst_idx = edge_index[1]
    h0, hidden, we2b = pl.pallas_call(
        _pre_body,
        out_shape=(jax.ShapeDtypeStruct((_N, _DP), _f32),
                   jax.ShapeDtypeStruct((_E, _H), jnp.bfloat16),
                   jax.ShapeDtypeStruct((_D * _D, _H), jnp.bfloat16)),
    )(x, W0, b0, edge_attr, We1, be1, We2)

    zeros_n = jnp.zeros((_N, _DP), _f32)
    ones_e = jnp.ones((_E, _DP), _f32)
    deg_parts = _sc_scatter_add(ones_e, dst_idx, zeros_n)

    msg_call = pl.pallas_call(
        _msg_body,
        grid=(_E // _EB,),
        in_specs=[
            pl.BlockSpec((_EB, _H), lambda i: (i, 0)),
            pl.BlockSpec((_EB, _DP), lambda i: (i, 0)),
            pl.BlockSpec((_D * _D, _H), lambda i: (0, 0)),
            pl.BlockSpec((_D, _D), lambda i: (0, 0)),
        ],
        out_specs=pl.BlockSpec((_EB, _DP), lambda i: (i, 0)),
        out_shape=jax.ShapeDtypeStruct((_E, _DP), _f32),
    )
    update_call = pl.pallas_call(
        _update_body,
        out_shape=jax.ShapeDtypeStruct((_N, _DP), _f32),
    )

    be2m = be2.reshape(_D, _D)
    h = h0
    for _ in range(_NUM_LAYER):
        xs = _sc_gather(h, src_idx)
        msg = msg_call(hidden, xs, we2b, be2m)
        aggr_parts = _sc_scatter_add(msg, dst_idx, zeros_n)
        h = update_call(aggr_parts, deg_parts, h, root, cbias,
                        gWih, gWhh, gbih, gbhh)

    return pl.pallas_call(
        _s2s_body,
        out_shape=jax.ShapeDtypeStruct((_G, 300), _f32),
    )(h, batch, lWih, lWhh, lbih, lbhh, W1, b1, W2, b2)


# speedup vs baseline: 4.4015x; 1.0222x over previous
import functools

import jax
import jax.numpy as jnp
from jax import lax
from jax.experimental import pallas as pl
from jax.experimental.pallas import tpu as pltpu
from jax.experimental.pallas import tpu_sc as plsc

_N = 4096
_E = 8192
_G = 256
_D = 64
_H = 128
_NUM_LAYER = 5
_STEPS = 3

_NW = 32
_EPW = _E // _NW
_CH = 128
_NCH = _EPW // _CH
_NPS = _N // 16
_DP = 128

_f32 = jnp.float32


@functools.cache
def _build_sc_gather():
    mesh = plsc.VectorSubcoreMesh(core_axis_name="c", subcore_axis_name="s")

    @functools.partial(
        pl.kernel,
        out_type=jax.ShapeDtypeStruct((_E, _DP), _f32),
        mesh=mesh,
        scratch_types=[
            pltpu.VMEM((_NCH, _CH), jnp.int32),
            pltpu.VMEM((_EPW, _DP), _f32),
            pltpu.SemaphoreType.DMA,
        ],
    )
    def gather(h_hbm, src_hbm, out_hbm, idx_v, rows_v, sem):
        wid = lax.axis_index("s") * 2 + lax.axis_index("c")
        base = wid * _EPW
        for j in range(_NCH):
            pltpu.sync_copy(src_hbm.at[pl.ds(base + j * _CH, _CH)],
                            idx_v.at[j])
            pltpu.async_copy(h_hbm.at[idx_v.at[j]],
                             rows_v.at[pl.ds(j * _CH, _CH)], sem).wait()
        pltpu.sync_copy(rows_v, out_hbm.at[pl.ds(base, _EPW)])

    return gather


@functools.cache
def _build_sc_scatter_add():
    mesh = plsc.VectorSubcoreMesh(core_axis_name="c", subcore_axis_name="s")

    @functools.partial(
        pl.kernel,
        out_type=jax.ShapeDtypeStruct((2, _N, _DP), _f32),
        mesh=mesh,
        scratch_types=[
            pltpu.VMEM((_NCH, _CH), jnp.int32),
            pltpu.VMEM((_EPW, _DP), _f32),
            pltpu.VMEM_SHARED((_N, _DP), _f32),
            pltpu.SemaphoreType.DMA,
        ],
    )
    def scatter_add(msg_hbm, dst_hbm, zero_hbm, out_hbm, idx_v, rows_v,
                    aggr_sh, sem):
        c = lax.axis_index("c")
        s = lax.axis_index("s")
        base = (s * 2 + c) * _EPW
        nb = s * _NPS
        pltpu.sync_copy(zero_hbm.at[pl.ds(nb, _NPS)],
                        aggr_sh.at[pl.ds(nb, _NPS)])
        pltpu.sync_copy(msg_hbm.at[pl.ds(base, _EPW)], rows_v)
        for j in range(_NCH):
            pltpu.sync_copy(dst_hbm.at[pl.ds(base + j * _CH, _CH)],
                            idx_v.at[j])
        plsc.subcore_barrier()
        for j in range(_NCH):
            pltpu.sync_copy(rows_v.at[pl.ds(j * _CH, _CH)],
                            aggr_sh.at[idx_v.at[j]], add=True)
        plsc.subcore_barrier()
        pltpu.sync_copy(aggr_sh.at[pl.ds(nb, _NPS)],
                        out_hbm.at[c, pl.ds(nb, _NPS)])

    return scatter_add


def _sc_gather(h, src):
    return _build_sc_gather()(h, src)


def _sc_scatter_add(msg, dst, zeros_n):
    return _build_sc_scatter_add()(msg, dst, zeros_n)


def _dotT(a, b):
    return lax.dot_general(a, b, (((1,), (1,)), ((), ())),
                           preferred_element_type=_f32)


def _pre_body(x_ref, w0_ref, b0_ref, ea_ref, we1_ref, be1_ref, we2_ref,
              h0_ref, hid_ref, we2b_ref):
    h0 = _dotT(x_ref[...], w0_ref[...]) + b0_ref[...][None, :]
    h0_ref[...] = jnp.concatenate(
        [jnp.maximum(h0, 0.0), jnp.zeros((_N, _DP - _D), _f32)], axis=1)
    hid = _dotT(ea_ref[...], we1_ref[...]) + be1_ref[...][None, :]
    hid_ref[...] = jnp.maximum(hid, 0.0).astype(jnp.bfloat16)
    we2b_ref[...] = we2_ref[...].astype(jnp.bfloat16)


_EB = 1024


def _msg_body(hid_ref, xs_ref, we2_ref, be2m_ref, msg_ref):
    wt = lax.dot_general(we2_ref[...], hid_ref[...], (((1,), (1,)), ((), ())),
                         preferred_element_type=_f32)
    xs = xs_ref[...][:, 0:_D]
    xst = xs.T
    acc = wt[0:_D, :] * xst[0:1, :]
    for i in range(1, _D):
        acc = acc + wt[i * _D:(i + 1) * _D, :] * xst[i:i + 1, :]
    msg = acc.T + jnp.dot(xs, be2m_ref[...], preferred_element_type=_f32)
    msg_ref[...] = jnp.concatenate(
        [msg, jnp.zeros((_EB, _DP - _D), _f32)], axis=1)


def _update_body(ap_ref, dp_ref, h_ref, root_ref, cb_ref, gwih_ref, gwhh_ref,
                 gbih_ref, gbhh_ref, out_ref):
    aggr = ap_ref[0][:, 0:_D] + ap_ref[1][:, 0:_D]
    deg = jnp.maximum(dp_ref[0][:, 0:_D] + dp_ref[1][:, 0:_D], 1.0)
    h = h_ref[...][:, 0:_D]
    m = jnp.maximum(aggr / deg
                    + jnp.dot(h, root_ref[...], preferred_element_type=_f32)
                    + cb_ref[...][None, :], 0.0)
    gi = _dotT(m, gwih_ref[...]) + gbih_ref[...][None, :]
    gh = _dotT(h, gwhh_ref[...]) + gbhh_ref[...][None, :]
    r = jax.nn.sigmoid(gi[:, :_D] + gh[:, :_D])
    z = jax.nn.sigmoid(gi[:, _D:2 * _D] + gh[:, _D:2 * _D])
    n = jnp.tanh(gi[:, 2 * _D:] + r * gh[:, 2 * _D:])
    out_ref[...] = jnp.concatenate(
        [(1.0 - z) * n + z * h, jnp.zeros((_N, _DP - _D), _f32)], axis=1)


def _s2s_body(h_ref, batch_ref, lwih_ref, lwhh_ref, lbih_ref, lbhh_ref,
              w1_ref, b1_ref, w2_ref, b2_ref, out_ref):
    out = h_ref[...][:, 0:_D]
    seg = lax.broadcasted_iota(jnp.int32, (_N, _G), 1)
    mb = batch_ref[...][:, None] == seg
    mf = jnp.where(mb, 1.0, 0.0)
    q_star = jnp.zeros((_G, 2 * _D), _f32)
    hh = jnp.zeros((_G, _D), _f32)
    cc = jnp.zeros((_G, _D), _f32)
    for _ in range(_STEPS):
        g = (_dotT(q_star, lwih_ref[...]) + lbih_ref[...][None, :]
             + _dotT(hh, lwhh_ref[...]) + lbhh_ref[...][None, :])
        ig = jax.nn.sigmoid(g[:, :_D])
        fg = jax.nn.sigmoid(g[:, _D:2 * _D])
        gg = jnp.tanh(g[:, 2 * _D:3 * _D])
        og = jax.nn.sigmoid(g[:, 3 * _D:])
        cc = fg * cc + ig * gg
        hh = og * jnp.tanh(cc)
        qb = jnp.dot(mf, hh, preferred_element_type=_f32)
        e = jnp.sum(out * qb, axis=1, keepdims=True)
        em = jnp.where(mb, e, -jnp.inf)
        emax = jnp.max(em, axis=0, keepdims=True)
        emax = jnp.where(jnp.isfinite(emax), emax, 0.0)
        eshift = e - jnp.sum(mf * emax, axis=1, keepdims=True)
        ew = jnp.exp(eshift)
        denom = lax.dot_general(mf, ew, (((0,), (0,)), ((), ())),
                                preferred_element_type=_f32)
        rden = 1.0 / jnp.maximum(denom, 1e-16)
        a = ew * jnp.dot(mf, rden, preferred_element_type=_f32)
        rvec = lax.dot_general(mf, a * out, (((0,), (0,)), ((), ())),
                               preferred_element_type=_f32)
        q_star = jnp.concatenate([hh, rvec], axis=1)
    o1 = _dotT(q_star, w1_ref[...]) + b1_ref[...][None, :]
    o2 = _dotT(o1, w2_ref[...]) + b2_ref[...][None, :]
    out_ref[...] = jnp.where(o2 > 0.0, o2, 0.01 * o2)


def kernel(x, edge_index, edge_attr, batch, W0, b0, We1, be1, We2, be2, root,
           cbias, gWih, gWhh, gbih, gbhh, lWih, lWhh, lbih, lbhh, W1, b1,
           W2, b2):
    src_idx = edge_index[0]
    dst_idx = edge_index[1]
    h0, hidden, we2b = pl.pallas_call(
        _pre_body,
        out_shape=(jax.ShapeDtypeStruct((_N, _DP), _f32),
                   jax.ShapeDtypeStruct((_E, _H), jnp.bfloat16),
                   jax.ShapeDtypeStruct((_D * _D, _H), jnp.bfloat16)),
    )(x, W0, b0, edge_attr, We1, be1, We2)

    zeros_n = jnp.zeros((_N, _DP), _f32)
    ones_e = jnp.ones((_E, _DP), _f32)
    deg_parts = _sc_scatter_add(ones_e, dst_idx, zeros_n)

    msg_call = pl.pallas_call(
        _msg_body,
        grid=(_E // _EB,),
        in_specs=[
            pl.BlockSpec((_EB, _H), lambda i: (i, 0)),
            pl.BlockSpec((_EB, _DP), lambda i: (i, 0)),
            pl.BlockSpec((_D * _D, _H), lambda i: (0, 0)),
            pl.BlockSpec((_D, _D), lambda i: (0, 0)),
        ],
        out_specs=pl.BlockSpec((_EB, _DP), lambda i: (i, 0)),
        out_shape=jax.ShapeDtypeStruct((_E, _DP), _f32),
    )
    update_call = pl.pallas_call(
        _update_body,
        out_shape=jax.ShapeDtypeStruct((_N, _DP), _f32),
    )

    be2m = be2.reshape(_D, _D)
    h = h0
    for _ in range(_NUM_LAYER):
        xs = _sc_gather(h, src_idx)
        msg = msg_call(hidden, xs, we2b, be2m)
        aggr_parts = _sc_scatter_add(msg, dst_idx, zeros_n)
        h = update_call(aggr_parts, deg_parts, h, root, cbias,
                        gWih, gWhh, gbih, gbhh)

    return pl.pallas_call(
        _s2s_body,
        out_shape=jax.ShapeDtypeStruct((_G, 300), _f32),
    )(h, batch, lWih, lWhh, lbih, lbhh, W1, b1, W2, b2)


# speedup vs baseline: 4.6564x; 1.0579x over previous
import functools

import jax
import jax.numpy as jnp
from jax import lax
from jax.experimental import pallas as pl
from jax.experimental.pallas import tpu as pltpu
from jax.experimental.pallas import tpu_sc as plsc

_N = 4096
_E = 8192
_G = 256
_D = 64
_H = 128
_NUM_LAYER = 5
_STEPS = 3

_NW = 32
_EPW = _E // _NW
_CH = 128
_NCH = _EPW // _CH
_NPS = _N // 16
_DP = 128

_f32 = jnp.float32


@functools.cache
def _build_sc_gather():
    mesh = plsc.VectorSubcoreMesh(core_axis_name="c", subcore_axis_name="s")

    @functools.partial(
        pl.kernel,
        out_type=jax.ShapeDtypeStruct((_E, _DP), _f32),
        mesh=mesh,
        scratch_types=[
            pltpu.VMEM((_NCH, _CH), jnp.int32),
            pltpu.VMEM((_EPW, _DP), _f32),
            [pltpu.SemaphoreType.DMA] * _NCH,
        ],
    )
    def gather(h_hbm, src_hbm, out_hbm, idx_v, rows_v, sems):
        wid = lax.axis_index("s") * 2 + lax.axis_index("c")
        base = wid * _EPW
        idx_cp = [pltpu.async_copy(src_hbm.at[pl.ds(base + j * _CH, _CH)],
                                   idx_v.at[j], sems[j])
                  for j in range(_NCH)]
        row_cp = []
        for j in range(_NCH):
            idx_cp[j].wait()
            row_cp.append(pltpu.async_copy(h_hbm.at[idx_v.at[j]],
                                           rows_v.at[pl.ds(j * _CH, _CH)],
                                           sems[j]))
        out_cp = []
        for j in range(_NCH):
            row_cp[j].wait()
            out_cp.append(pltpu.async_copy(
                rows_v.at[pl.ds(j * _CH, _CH)],
                out_hbm.at[pl.ds(base + j * _CH, _CH)], sems[j]))
        for cp in out_cp:
            cp.wait()

    return gather


@functools.cache
def _build_sc_scatter_add():
    mesh = plsc.VectorSubcoreMesh(core_axis_name="c", subcore_axis_name="s")

    @functools.partial(
        pl.kernel,
        out_type=jax.ShapeDtypeStruct((2, _N, _DP), _f32),
        mesh=mesh,
        scratch_types=[
            pltpu.VMEM((_NCH, _CH), jnp.int32),
            pltpu.VMEM((_EPW, _DP), _f32),
            pltpu.VMEM_SHARED((_N, _DP), _f32),
            [pltpu.SemaphoreType.DMA] * 3,
        ],
    )
    def scatter_add(msg_hbm, dst_hbm, zero_hbm, out_hbm, idx_v, rows_v,
                    aggr_sh, sems):
        c = lax.axis_index("c")
        s = lax.axis_index("s")
        base = (s * 2 + c) * _EPW
        nb = s * _NPS
        z_cp = pltpu.async_copy(zero_hbm.at[pl.ds(nb, _NPS)],
                                aggr_sh.at[pl.ds(nb, _NPS)], sems[0])
        m_cp = pltpu.async_copy(msg_hbm.at[pl.ds(base, _EPW)], rows_v,
                                sems[1])
        i_cp = [pltpu.async_copy(dst_hbm.at[pl.ds(base + j * _CH, _CH)],
                                 idx_v.at[j], sems[2])
                for j in range(_NCH)]
        z_cp.wait()
        m_cp.wait()
        for cp in i_cp:
            cp.wait()
        plsc.subcore_barrier()
        for j in range(_NCH):
            pltpu.sync_copy(rows_v.at[pl.ds(j * _CH, _CH)],
                            aggr_sh.at[idx_v.at[j]], add=True)
        plsc.subcore_barrier()
        pltpu.sync_copy(aggr_sh.at[pl.ds(nb, _NPS)],
                        out_hbm.at[c, pl.ds(nb, _NPS)])

    return scatter_add


def _sc_gather(h, src):
    return _build_sc_gather()(h, src)


def _sc_scatter_add(msg, dst, zeros_n):
    return _build_sc_scatter_add()(msg, dst, zeros_n)


def _dotT(a, b):
    return lax.dot_general(a, b, (((1,), (1,)), ((), ())),
                           preferred_element_type=_f32)


def _pre_body(x_ref, w0_ref, b0_ref, ea_ref, we1_ref, be1_ref, we2_ref,
              h0_ref, hid_ref, we2b_ref, zn_ref, oe_ref):
    h0 = _dotT(x_ref[...], w0_ref[...]) + b0_ref[...][None, :]
    h0_ref[...] = jnp.concatenate(
        [jnp.maximum(h0, 0.0), jnp.zeros((_N, _DP - _D), _f32)], axis=1)
    hid = _dotT(ea_ref[...], we1_ref[...]) + be1_ref[...][None, :]
    hid_ref[...] = jnp.maximum(hid, 0.0).astype(jnp.bfloat16)
    we2b_ref[...] = we2_ref[...].astype(jnp.bfloat16)
    zn_ref[...] = jnp.zeros((_N, _DP), _f32)
    oe_ref[...] = jnp.ones((_E, _DP), _f32)


_EB = 1024


def _msg_body(hid_ref, xs_ref, we2_ref, be2m_ref, msg_ref):
    wt = lax.dot_general(we2_ref[...], hid_ref[...], (((1,), (1,)), ((), ())),
                         preferred_element_type=_f32)
    xs = xs_ref[...][:, 0:_D]
    xst = xs.T
    acc = wt[0:_D, :] * xst[0:1, :]
    for i in range(1, _D):
        acc = acc + wt[i * _D:(i + 1) * _D, :] * xst[i:i + 1, :]
    msg = acc.T + jnp.dot(xs, be2m_ref[...], preferred_element_type=_f32)
    msg_ref[...] = jnp.concatenate(
        [msg, jnp.zeros((_EB, _DP - _D), _f32)], axis=1)


def _update_body(ap_ref, dp_ref, h_ref, root_ref, cb_ref, wg_ref, bg_ref,
                 out_ref):
    aggr = ap_ref[0][:, 0:_D] + ap_ref[1][:, 0:_D]
    deg = jnp.maximum(dp_ref[0][:, 0:_D] + dp_ref[1][:, 0:_D], 1.0)
    h = h_ref[...][:, 0:_D]
    m = jnp.maximum(aggr / deg
                    + jnp.dot(h, root_ref[...], preferred_element_type=_f32)
                    + cb_ref[...][None, :], 0.0)
    g = (lax.dot_general(jnp.concatenate([m, h], axis=1), wg_ref[...],
                         (((1,), (1,)), ((), ())),
                         preferred_element_type=_f32)
         + bg_ref[...][None, :])
    r = jax.nn.sigmoid(g[:, :_D])
    z = jax.nn.sigmoid(g[:, _D:2 * _D])
    n = jnp.tanh(g[:, 2 * _D:3 * _D] + r * g[:, 3 * _D:])
    out_ref[...] = jnp.concatenate(
        [(1.0 - z) * n + z * h, jnp.zeros((_N, _DP - _D), _f32)], axis=1)


def _s2s_body(h_ref, batch_ref, lwih_ref, lwhh_ref, lbih_ref, lbhh_ref,
              w1_ref, b1_ref, w2_ref, b2_ref, out_ref):
    out = h_ref[...][:, 0:_D]
    seg = lax.broadcasted_iota(jnp.int32, (_N, _G), 1)
    mb = batch_ref[...][:, None] == seg
    mf = jnp.where(mb, 1.0, 0.0)
    q_star = jnp.zeros((_G, 2 * _D), _f32)
    hh = jnp.zeros((_G, _D), _f32)
    cc = jnp.zeros((_G, _D), _f32)
    for _ in range(_STEPS):
        g = (_dotT(q_star, lwih_ref[...]) + lbih_ref[...][None, :]
             + _dotT(hh, lwhh_ref[...]) + lbhh_ref[...][None, :])
        ig = jax.nn.sigmoid(g[:, :_D])
        fg = jax.nn.sigmoid(g[:, _D:2 * _D])
        gg = jnp.tanh(g[:, 2 * _D:3 * _D])
        og = jax.nn.sigmoid(g[:, 3 * _D:])
        cc = fg * cc + ig * gg
        hh = og * jnp.tanh(cc)
        qb = jnp.dot(mf, hh, preferred_element_type=_f32)
        e = jnp.sum(out * qb, axis=1, keepdims=True)
        em = jnp.where(mb, e, -jnp.inf)
        emax = jnp.max(em, axis=0, keepdims=True)
        emax = jnp.where(jnp.isfinite(emax), emax, 0.0)
        eshift = e - jnp.sum(mf * emax, axis=1, keepdims=True)
        ew = jnp.exp(eshift)
        denom = lax.dot_general(mf, ew, (((0,), (0,)), ((), ())),
                                preferred_element_type=_f32)
        rden = 1.0 / jnp.maximum(denom, 1e-16)
        a = ew * jnp.dot(mf, rden, preferred_element_type=_f32)
        rvec = lax.dot_general(mf, a * out, (((0,), (0,)), ((), ())),
                               preferred_element_type=_f32)
        q_star = jnp.concatenate([hh, rvec], axis=1)
    o1 = _dotT(q_star, w1_ref[...]) + b1_ref[...][None, :]
    o2 = _dotT(o1, w2_ref[...]) + b2_ref[...][None, :]
    out_ref[...] = jnp.where(o2 > 0.0, o2, 0.01 * o2)


def kernel(x, edge_index, edge_attr, batch, W0, b0, We1, be1, We2, be2, root,
           cbias, gWih, gWhh, gbih, gbhh, lWih, lWhh, lbih, lbhh, W1, b1,
           W2, b2):
    src_idx = edge_index[0]
    dst_idx = edge_index[1]
    h0, hidden, we2b, zeros_n, ones_e = pl.pallas_call(
        _pre_body,
        out_shape=(jax.ShapeDtypeStruct((_N, _DP), _f32),
                   jax.ShapeDtypeStruct((_E, _H), jnp.bfloat16),
                   jax.ShapeDtypeStruct((_D * _D, _H), jnp.bfloat16),
                   jax.ShapeDtypeStruct((_N, _DP), _f32),
                   jax.ShapeDtypeStruct((_E, _DP), _f32)),
    )(x, W0, b0, edge_attr, We1, be1, We2)

    deg_parts = _sc_scatter_add(ones_e, dst_idx, zeros_n)

    z64 = jnp.zeros((_D, _D), _f32)
    wg = jnp.concatenate([
        jnp.concatenate([gWih[:2 * _D], gWhh[:2 * _D]], axis=1),
        jnp.concatenate([gWih[2 * _D:], z64], axis=1),
        jnp.concatenate([z64, gWhh[2 * _D:]], axis=1),
    ], axis=0)
    bg = jnp.concatenate([gbih[:2 * _D] + gbhh[:2 * _D],
                          gbih[2 * _D:], gbhh[2 * _D:]])

    msg_call = pl.pallas_call(
        _msg_body,
        grid=(_E // _EB,),
        in_specs=[
            pl.BlockSpec((_EB, _H), lambda i: (i, 0)),
            pl.BlockSpec((_EB, _DP), lambda i: (i, 0)),
            pl.BlockSpec((_D * _D, _H), lambda i: (0, 0)),
            pl.BlockSpec((_D, _D), lambda i: (0, 0)),
        ],
        out_specs=pl.BlockSpec((_EB, _DP), lambda i: (i, 0)),
        out_shape=jax.ShapeDtypeStruct((_E, _DP), _f32),
    )
    update_call = pl.pallas_call(
        _update_body,
        out_shape=jax.ShapeDtypeStruct((_N, _DP), _f32),
    )

    be2m = be2.reshape(_D, _D)
    h = h0
    for _ in range(_NUM_LAYER):
        xs = _sc_gather(h, src_idx)
        msg = msg_call(hidden, xs, we2b, be2m)
        aggr_parts = _sc_scatter_add(msg, dst_idx, zeros_n)
        h = update_call(aggr_parts, deg_parts, h, root, cbias, wg, bg)

    return pl.pallas_call(
        _s2s_body,
        out_shape=jax.ShapeDtypeStruct((_G, 300), _f32),
    )(h, batch, lWih, lWhh, lbih, lbhh, W1, b1, W2, b2)
